# baseline (speedup 1.0000x reference)
import jax
import jax.numpy as jnp
from jax import lax
from jax.experimental import pallas as pl
from jax.experimental.pallas import tpu as pltpu
from jax.experimental.pallas import tpu_sc as plsc

V = 1000000
D = 64
B = 16384
NEG = 20
NP = NEG + 1

NC = 2
NS = 16
NW = NC * NS
B_PER_W = B // NW
BLK = 32
NBLK = B_PER_W // BLK
NG = BLK // 16

_CH = 1024
_NCH = (V + _CH - 1) // _CH


def _pad_body(x_ref, o_ref):
    xt = jnp.transpose(x_ref[...], (1, 0))
    o_ref[...] = jnp.concatenate([xt, jnp.zeros((_CH, D), jnp.float32)], axis=1)


@jax.jit
def _tc_pad(tab_T):
    return pl.pallas_call(
        _pad_body,
        grid=(_NCH,),
        in_specs=[pl.BlockSpec((D, _CH), lambda i: (0, i))],
        out_specs=pl.BlockSpec((_CH, 2 * D), lambda i: (i, 0)),
        out_shape=jax.ShapeDtypeStruct((V, 2 * D), jnp.float32),
    )(tab_T)


def _sc_body(cw_hbm, xw_hbm, nT_hbm, in_hbm, out_hbm,
             pos_hbm, negT_hbm,
             cidx, xidx, nidx, crows, prows, scores, sem):
    wid = lax.axis_index("s") * NC + lax.axis_index("c")
    wbase = wid * B_PER_W
    lanes = lax.iota(jnp.int32, 16)

    pltpu.sync_copy(cw_hbm.at[pl.ds(wbase, B_PER_W)], cidx)
    pltpu.sync_copy(xw_hbm.at[pl.ds(wbase, B_PER_W)], xidx)
    for j in range(NEG):
        pltpu.sync_copy(nT_hbm.at[j, pl.ds(wbase, B_PER_W)], nidx.at[j])

    def blk_body(sb, _):
        off = sb * BLK
        descs = [
            pltpu.async_copy(in_hbm.at[cidx.at[pl.ds(off, BLK)]], crows, sem),
            pltpu.async_copy(out_hbm.at[xidx.at[pl.ds(off, BLK)]], prows.at[0], sem),
        ]
        for j in range(NEG):
            descs.append(
                pltpu.async_copy(
                    out_hbm.at[nidx.at[j, pl.ds(off, BLK)]], prows.at[1 + j], sem
                )
            )
        for dsc in descs:
            dsc.wait()

        def group_body(g, _):
            rid = g * 16 + lanes

            def d_body(d, accs):
                dv = jnp.broadcast_to(d, (16,))
                cv = plsc.load_gather(crows, [rid, dv])
                return tuple(
                    acc
                    + cv
                    * plsc.load_gather(
                        prows, [jnp.full((16,), t, jnp.int32), rid, dv]
                    )
                    for t, acc in enumerate(accs)
                )

            accs = lax.fori_loop(
                0, D, d_body, tuple(jnp.zeros((16,), jnp.float32) for _ in range(NP))
            )
            for t in range(NP):
                scores[t, pl.ds(g * 16, 16)] = accs[t]
            return _

        lax.fori_loop(0, NG, group_body, None)

        pltpu.sync_copy(scores.at[0], pos_hbm.at[pl.ds(wbase + off, BLK)])
        for j in range(NEG):
            pltpu.sync_copy(scores.at[1 + j], negT_hbm.at[j, pl.ds(wbase + off, BLK)])
        return _

    lax.fori_loop(0, NBLK, blk_body, None)


@jax.jit
def _sc_scores(center_words, context_words, neg_T, in_p, out_p):
    mesh = plsc.VectorSubcoreMesh(
        core_axis_name="c", subcore_axis_name="s", num_cores=NC, num_subcores=NS
    )
    f = pl.kernel(
        _sc_body,
        out_type=(
            jax.ShapeDtypeStruct((B,), jnp.float32),
            jax.ShapeDtypeStruct((NEG, B), jnp.float32),
        ),
        mesh=mesh,
        compiler_params=pltpu.CompilerParams(needs_layout_passes=False),
        scratch_types=[
            pltpu.VMEM((B_PER_W,), jnp.int32),
            pltpu.VMEM((B_PER_W,), jnp.int32),
            pltpu.VMEM((NEG, B_PER_W), jnp.int32),
            pltpu.VMEM((BLK, 2 * D), jnp.float32),
            pltpu.VMEM((NP, BLK, 2 * D), jnp.float32),
            pltpu.VMEM((NP, BLK), jnp.float32),
            pltpu.SemaphoreType.DMA,
        ],
    )
    return f(center_words, context_words, neg_T, in_p, out_p)


def _loss_body(pos_ref, neg_ref, out_ref):
    p = pos_ref[...]
    n = neg_ref[...]
    total = jnp.sum(jax.nn.log_sigmoid(p)) + jnp.sum(jax.nn.log_sigmoid(-n))
    out_ref[...] = jnp.reshape(-total / B, (1, 1))


@jax.jit
def _tc_loss(pos, neg):
    out = pl.pallas_call(
        _loss_body,
        out_shape=jax.ShapeDtypeStruct((1, 1), jnp.float32),
    )(pos.reshape(128, 128), neg.reshape(NEG * B // 128, 128))
    return out[0, 0]


def kernel(center_words, context_words, negative_words, in_emb, out_emb):
    in_p = _tc_pad(in_emb.T)
    out_p = _tc_pad(out_emb.T)
    neg_T = negative_words.T
    pos, negs = _sc_scores(center_words, context_words, neg_T, in_p, out_p)
    return _tc_loss(pos, negs)


# speedup vs baseline: 1.3007x; 1.3007x over previous
import jax
import jax.numpy as jnp
from jax import lax
from jax.experimental import pallas as pl
from jax.experimental.pallas import tpu as pltpu
from jax.experimental.pallas import tpu_sc as plsc

V = 1000000
D = 64
B = 16384
NEG = 20
NP = NEG + 1

NC = 2
NS = 16
NW = NC * NS
B_PER_W = B // NW
BLK = 32
NBLK = B_PER_W // BLK
NG = BLK // 16

_CH = 2048
_NCH = (V + _CH - 1) // _CH


def _pad_body(x_ref, o_ref):
    proj = (
        lax.broadcasted_iota(jnp.int32, (D, 2 * D), 0)
        == lax.broadcasted_iota(jnp.int32, (D, 2 * D), 1)
    ).astype(jnp.float32)
    o_ref[...] = lax.dot_general(
        x_ref[...], proj,
        dimension_numbers=(((0,), (0,)), ((), ())),
        preferred_element_type=jnp.float32,
    )


@jax.jit
def _tc_pad(tab_T):
    return pl.pallas_call(
        _pad_body,
        grid=(_NCH,),
        in_specs=[pl.BlockSpec((D, _CH), lambda i: (0, i))],
        out_specs=pl.BlockSpec((_CH, 2 * D), lambda i: (i, 0)),
        out_shape=jax.ShapeDtypeStruct((V, 2 * D), jnp.float32),
    )(tab_T)


def _sc_body(cw_hbm, xw_hbm, nT_hbm, in_hbm, out_hbm,
             pos_hbm, negT_hbm,
             cidx, xidx, nidx, crows, prows, scores, sem):
    wid = lax.axis_index("s") * NC + lax.axis_index("c")
    wbase = wid * B_PER_W
    lanes = lax.iota(jnp.int32, 16)

    pltpu.sync_copy(cw_hbm.at[pl.ds(wbase, B_PER_W)], cidx)
    pltpu.sync_copy(xw_hbm.at[pl.ds(wbase, B_PER_W)], xidx)
    for j in range(NEG):
        pltpu.sync_copy(nT_hbm.at[j, pl.ds(wbase, B_PER_W)], nidx.at[j])

    def blk_body(sb, _):
        off = sb * BLK
        descs = [
            pltpu.async_copy(in_hbm.at[cidx.at[pl.ds(off, BLK)]], crows, sem),
            pltpu.async_copy(out_hbm.at[xidx.at[pl.ds(off, BLK)]], prows.at[0], sem),
        ]
        for j in range(NEG):
            descs.append(
                pltpu.async_copy(
                    out_hbm.at[nidx.at[j, pl.ds(off, BLK)]], prows.at[1 + j], sem
                )
            )
        for dsc in descs:
            dsc.wait()

        def group_body(g, _):
            rid = g * 16 + lanes

            def d_body(d, accs):
                dv = jnp.broadcast_to(d, (16,))
                cv = plsc.load_gather(crows, [rid, dv])
                return tuple(
                    acc
                    + cv
                    * plsc.load_gather(
                        prows, [jnp.full((16,), t, jnp.int32), rid, dv]
                    )
                    for t, acc in enumerate(accs)
                )

            accs = lax.fori_loop(
                0, D, d_body, tuple(jnp.zeros((16,), jnp.float32) for _ in range(NP))
            )
            for t in range(NP):
                scores[t, pl.ds(g * 16, 16)] = accs[t]
            return _

        lax.fori_loop(0, NG, group_body, None)

        pltpu.sync_copy(scores.at[0], pos_hbm.at[pl.ds(wbase + off, BLK)])
        for j in range(NEG):
            pltpu.sync_copy(scores.at[1 + j], negT_hbm.at[j, pl.ds(wbase + off, BLK)])
        return _

    lax.fori_loop(0, NBLK, blk_body, None)


@jax.jit
def _sc_scores(center_words, context_words, neg_T, in_p, out_p):
    mesh = plsc.VectorSubcoreMesh(
        core_axis_name="c", subcore_axis_name="s", num_cores=NC, num_subcores=NS
    )
    f = pl.kernel(
        _sc_body,
        out_type=(
            jax.ShapeDtypeStruct((B,), jnp.float32),
            jax.ShapeDtypeStruct((NEG, B), jnp.float32),
        ),
        mesh=mesh,
        compiler_params=pltpu.CompilerParams(needs_layout_passes=False),
        scratch_types=[
            pltpu.VMEM((B_PER_W,), jnp.int32),
            pltpu.VMEM((B_PER_W,), jnp.int32),
            pltpu.VMEM((NEG, B_PER_W), jnp.int32),
            pltpu.VMEM((BLK, 2 * D), jnp.float32),
            pltpu.VMEM((NP, BLK, 2 * D), jnp.float32),
            pltpu.VMEM((NP, BLK), jnp.float32),
            pltpu.SemaphoreType.DMA,
        ],
    )
    return f(center_words, context_words, neg_T, in_p, out_p)


def _loss_body(pos_ref, neg_ref, out_ref):
    p = pos_ref[...]
    n = neg_ref[...]
    total = jnp.sum(jax.nn.log_sigmoid(p)) + jnp.sum(jax.nn.log_sigmoid(-n))
    out_ref[...] = jnp.reshape(-total / B, (1, 1))


@jax.jit
def _tc_loss(pos, neg):
    out = pl.pallas_call(
        _loss_body,
        out_shape=jax.ShapeDtypeStruct((1, 1), jnp.float32),
    )(pos.reshape(128, 128), neg.reshape(NEG * B // 128, 128))
    return out[0, 0]


def kernel(center_words, context_words, negative_words, in_emb, out_emb):
    in_p = _tc_pad(in_emb.T)
    out_p = _tc_pad(out_emb.T)
    neg_T = negative_words.T
    pos, negs = _sc_scores(center_words, context_words, neg_T, in_p, out_p)
    return _tc_loss(pos, negs)


# speedup vs baseline: 1.3694x; 1.0528x over previous
import jax
import jax.numpy as jnp
from jax import lax
from jax.experimental import pallas as pl
from jax.experimental.pallas import tpu as pltpu
from jax.experimental.pallas import tpu_sc as plsc

V = 1000000
D = 64
B = 16384
NEG = 20
NP = NEG + 1

NC = 2
NS = 16
NW = NC * NS
B_PER_W = B // NW
BLK = 32
NBLK = B_PER_W // BLK
NG = BLK // 16

_CH = 2048
_NCH = (V + _CH - 1) // _CH
_VP = _NCH * _CH // 2


def _pad_body(x_ref, o_ref):
    proj = (
        lax.broadcasted_iota(jnp.int32, (D, D), 0)
        == lax.broadcasted_iota(jnp.int32, (D, D), 1)
    ).astype(jnp.float32)
    xt = lax.dot_general(
        x_ref[...], proj,
        dimension_numbers=(((0,), (0,)), ((), ())),
        preferred_element_type=jnp.float32,
    )
    o_ref[...] = jnp.concatenate([xt[: _CH // 2], xt[_CH // 2 :]], axis=1)


@jax.jit
def _tc_pad(tab_T):
    return pl.pallas_call(
        _pad_body,
        grid=(_NCH,),
        in_specs=[pl.BlockSpec((D, _CH), lambda i: (0, i))],
        out_specs=pl.BlockSpec((_CH // 2, 2 * D), lambda i: (i, 0)),
        out_shape=jax.ShapeDtypeStruct((_VP, 2 * D), jnp.float32),
    )(tab_T)


def _sc_body(cw_hbm, xw_hbm, nT_hbm, in_hbm, out_hbm,
             pos_hbm, negT_hbm,
             cidx, xidx, nidx, cpk, xpk, npk, crows, prows, scores, sem):
    wid = lax.axis_index("s") * NC + lax.axis_index("c")
    wbase = wid * B_PER_W
    lanes = lax.iota(jnp.int32, 16)

    pltpu.sync_copy(cw_hbm.at[pl.ds(wbase, B_PER_W)], cidx)
    pltpu.sync_copy(xw_hbm.at[pl.ds(wbase, B_PER_W)], xidx)
    for j in range(NEG):
        pltpu.sync_copy(nT_hbm.at[j, pl.ds(wbase, B_PER_W)], nidx.at[j])

    def pk_body(k, _):
        s = pl.ds(k * 16, 16)
        c, x = cidx[s], xidx[s]
        cpk[s] = ((c >> 11) << 10) + (c & 1023)
        xpk[s] = ((x >> 11) << 10) + (x & 1023)
        for j in range(NEG):
            nj = nidx[j, s]
            npk[j, s] = ((nj >> 11) << 10) + (nj & 1023)
        return _

    lax.fori_loop(0, B_PER_W // 16, pk_body, None)

    def blk_body(sb, _):
        off = sb * BLK
        descs = [
            pltpu.async_copy(in_hbm.at[cpk.at[pl.ds(off, BLK)]], crows, sem),
            pltpu.async_copy(out_hbm.at[xpk.at[pl.ds(off, BLK)]], prows.at[0], sem),
        ]
        for j in range(NEG):
            descs.append(
                pltpu.async_copy(
                    out_hbm.at[npk.at[j, pl.ds(off, BLK)]], prows.at[1 + j], sem
                )
            )
        for dsc in descs:
            dsc.wait()

        def group_body(g, _):
            rid = g * 16 + lanes
            s = pl.ds(off + g * 16, 16)
            cbase = ((cidx[s] >> 10) & 1) * D
            pbase = [((xidx[s] >> 10) & 1) * D] + [
                ((nidx[j, s] >> 10) & 1) * D for j in range(NEG)
            ]

            def d_body(d, accs):
                cv = plsc.load_gather(crows, [rid, cbase + d])
                return tuple(
                    acc
                    + cv
                    * plsc.load_gather(
                        prows, [jnp.full((16,), t, jnp.int32), rid, pbase[t] + d]
                    )
                    for t, acc in enumerate(accs)
                )

            accs = lax.fori_loop(
                0, D, d_body, tuple(jnp.zeros((16,), jnp.float32) for _ in range(NP))
            )
            for t in range(NP):
                scores[t, pl.ds(g * 16, 16)] = accs[t]
            return _

        lax.fori_loop(0, NG, group_body, None)

        pltpu.sync_copy(scores.at[0], pos_hbm.at[pl.ds(wbase + off, BLK)])
        for j in range(NEG):
            pltpu.sync_copy(scores.at[1 + j], negT_hbm.at[j, pl.ds(wbase + off, BLK)])
        return _

    lax.fori_loop(0, NBLK, blk_body, None)


@jax.jit
def _sc_scores(center_words, context_words, neg_T, in_p, out_p):
    mesh = plsc.VectorSubcoreMesh(
        core_axis_name="c", subcore_axis_name="s", num_cores=NC, num_subcores=NS
    )
    f = pl.kernel(
        _sc_body,
        out_type=(
            jax.ShapeDtypeStruct((B,), jnp.float32),
            jax.ShapeDtypeStruct((NEG, B), jnp.float32),
        ),
        mesh=mesh,
        compiler_params=pltpu.CompilerParams(needs_layout_passes=False),
        scratch_types=[
            pltpu.VMEM((B_PER_W,), jnp.int32),
            pltpu.VMEM((B_PER_W,), jnp.int32),
            pltpu.VMEM((NEG, B_PER_W), jnp.int32),
            pltpu.VMEM((B_PER_W,), jnp.int32),
            pltpu.VMEM((B_PER_W,), jnp.int32),
            pltpu.VMEM((NEG, B_PER_W), jnp.int32),
            pltpu.VMEM((BLK, 2 * D), jnp.float32),
            pltpu.VMEM((NP, BLK, 2 * D), jnp.float32),
            pltpu.VMEM((NP, BLK), jnp.float32),
            pltpu.SemaphoreType.DMA,
        ],
    )
    return f(center_words, context_words, neg_T, in_p, out_p)


def _loss_body(pos_ref, neg_ref, out_ref):
    p = pos_ref[...]
    n = neg_ref[...]
    total = jnp.sum(jax.nn.log_sigmoid(p)) + jnp.sum(jax.nn.log_sigmoid(-n))
    out_ref[...] = jnp.reshape(-total / B, (1, 1))


@jax.jit
def _tc_loss(pos, neg):
    out = pl.pallas_call(
        _loss_body,
        out_shape=jax.ShapeDtypeStruct((1, 1), jnp.float32),
    )(pos.reshape(128, 128), neg.reshape(NEG * B // 128, 128))
    return out[0, 0]


def kernel(center_words, context_words, negative_words, in_emb, out_emb):
    in_p = _tc_pad(in_emb.T)
    out_p = _tc_pad(out_emb.T)
    neg_T = negative_words.T
    pos, negs = _sc_scores(center_words, context_words, neg_T, in_p, out_p)
    return _tc_loss(pos, negs)


# speedup vs baseline: 1.6811x; 1.2276x over previous
import jax
import jax.numpy as jnp
from jax import lax
from jax.experimental import pallas as pl
from jax.experimental.pallas import tpu as pltpu
from jax.experimental.pallas import tpu_sc as plsc

V = 1000000
D = 64
B = 16384
NEG = 20
NP = NEG + 1

NC = 2
NS = 16
NW = NC * NS
B_PER_W = B // NW
BLK = 32
NBLK = B_PER_W // BLK
NG = BLK // 16

_CH = 4096
_NCH = (V + _CH - 1) // _CH
_VP = _NCH * _CH // 2
_HALF = _CH // 2
_SH_CH = _CH.bit_length() - 1
_SH_H = _HALF.bit_length() - 1


def _pad_body(x_ref, o_ref):
    proj = (
        lax.broadcasted_iota(jnp.int32, (D, D), 0)
        == lax.broadcasted_iota(jnp.int32, (D, D), 1)
    ).astype(jnp.float32)
    xt = lax.dot_general(
        x_ref[...], proj,
        dimension_numbers=(((0,), (0,)), ((), ())),
        preferred_element_type=jnp.float32,
    )
    o_ref[...] = jnp.concatenate([xt[: _CH // 2], xt[_CH // 2 :]], axis=1)


@jax.jit
def _tc_pad(tab_T):
    return pl.pallas_call(
        _pad_body,
        grid=(_NCH,),
        in_specs=[pl.BlockSpec((D, _CH), lambda i: (0, i))],
        out_specs=pl.BlockSpec((_CH // 2, 2 * D), lambda i: (i, 0)),
        out_shape=jax.ShapeDtypeStruct((_VP, 2 * D), jnp.float32),
    )(tab_T)


def _sc_body(cw_hbm, xw_hbm, nT_hbm, in_hbm, out_hbm,
             pos_hbm, negT_hbm,
             cidx, xidx, nidx, cpk, xpk, npk, crows, prows, scores, sem):
    wid = lax.axis_index("s") * NC + lax.axis_index("c")
    wbase = wid * B_PER_W
    lanes = lax.iota(jnp.int32, 16)

    pltpu.sync_copy(cw_hbm.at[pl.ds(wbase, B_PER_W)], cidx)
    pltpu.sync_copy(xw_hbm.at[pl.ds(wbase, B_PER_W)], xidx)
    for j in range(NEG):
        pltpu.sync_copy(nT_hbm.at[j, pl.ds(wbase, B_PER_W)], nidx.at[j])

    def pk_body(k, _):
        s = pl.ds(k * 16, 16)
        c, x = cidx[s], xidx[s]
        cpk[s] = ((c >> _SH_CH) << _SH_H) + (c & (_HALF - 1))
        xpk[s] = ((x >> _SH_CH) << _SH_H) + (x & (_HALF - 1))
        for j in range(NEG):
            nj = nidx[j, s]
            npk[j, s] = ((nj >> _SH_CH) << _SH_H) + (nj & (_HALF - 1))
        return _

    lax.fori_loop(0, B_PER_W // 16, pk_body, None)

    def blk_body(sb, _):
        off = sb * BLK
        descs = [
            pltpu.async_copy(in_hbm.at[cpk.at[pl.ds(off, BLK)]], crows, sem),
            pltpu.async_copy(out_hbm.at[xpk.at[pl.ds(off, BLK)]], prows.at[0], sem),
        ]
        for j in range(NEG):
            descs.append(
                pltpu.async_copy(
                    out_hbm.at[npk.at[j, pl.ds(off, BLK)]], prows.at[1 + j], sem
                )
            )
        for dsc in descs:
            dsc.wait()

        def group_body(g, _):
            rid = g * 16 + lanes
            s = pl.ds(off + g * 16, 16)
            cbase = ((cidx[s] >> _SH_H) & 1) * D
            pbase = [((xidx[s] >> _SH_H) & 1) * D] + [
                ((nidx[j, s] >> _SH_H) & 1) * D for j in range(NEG)
            ]

            def d_body(d, accs):
                cv = plsc.load_gather(crows, [rid, cbase + d])
                return tuple(
                    acc
                    + cv
                    * plsc.load_gather(
                        prows, [jnp.full((16,), t, jnp.int32), rid, pbase[t] + d]
                    )
                    for t, acc in enumerate(accs)
                )

            accs = lax.fori_loop(
                0, D, d_body, tuple(jnp.zeros((16,), jnp.float32) for _ in range(NP))
            )
            for t in range(NP):
                scores[t, pl.ds(g * 16, 16)] = accs[t]
            return _

        lax.fori_loop(0, NG, group_body, None)

        pltpu.sync_copy(scores.at[0], pos_hbm.at[pl.ds(wbase + off, BLK)])
        for j in range(NEG):
            pltpu.sync_copy(scores.at[1 + j], negT_hbm.at[j, pl.ds(wbase + off, BLK)])
        return _

    lax.fori_loop(0, NBLK, blk_body, None)


@jax.jit
def _sc_scores(center_words, context_words, neg_T, in_p, out_p):
    mesh = plsc.VectorSubcoreMesh(
        core_axis_name="c", subcore_axis_name="s", num_cores=NC, num_subcores=NS
    )
    f = pl.kernel(
        _sc_body,
        out_type=(
            jax.ShapeDtypeStruct((B,), jnp.float32),
            jax.ShapeDtypeStruct((NEG, B), jnp.float32),
        ),
        mesh=mesh,
        compiler_params=pltpu.CompilerParams(needs_layout_passes=False),
        scratch_types=[
            pltpu.VMEM((B_PER_W,), jnp.int32),
            pltpu.VMEM((B_PER_W,), jnp.int32),
            pltpu.VMEM((NEG, B_PER_W), jnp.int32),
            pltpu.VMEM((B_PER_W,), jnp.int32),
            pltpu.VMEM((B_PER_W,), jnp.int32),
            pltpu.VMEM((NEG, B_PER_W), jnp.int32),
            pltpu.VMEM((BLK, 2 * D), jnp.float32),
            pltpu.VMEM((NP, BLK, 2 * D), jnp.float32),
            pltpu.VMEM((NP, BLK), jnp.float32),
            pltpu.SemaphoreType.DMA,
        ],
    )
    return f(center_words, context_words, neg_T, in_p, out_p)


def _loss_body(pos_ref, neg_ref, out_ref):
    p = pos_ref[...]
    n = neg_ref[...]
    total = jnp.sum(jax.nn.log_sigmoid(p)) + jnp.sum(jax.nn.log_sigmoid(-n))
    out_ref[...] = jnp.reshape(-total / B, (1, 1))


@jax.jit
def _tc_loss(pos, neg):
    out = pl.pallas_call(
        _loss_body,
        out_shape=jax.ShapeDtypeStruct((1, 1), jnp.float32),
    )(pos.reshape(128, 128), neg.reshape(NEG * B // 128, 128))
    return out[0, 0]


def kernel(center_words, context_words, negative_words, in_emb, out_emb):
    in_p = _tc_pad(in_emb.T)
    out_p = _tc_pad(out_emb.T)
    neg_T = negative_words.T
    pos, negs = _sc_scores(center_words, context_words, neg_T, in_p, out_p)
    return _tc_loss(pos, negs)


# speedup vs baseline: 1.9091x; 1.1356x over previous
import jax
import jax.numpy as jnp
from jax import lax
from jax.experimental import pallas as pl
from jax.experimental.pallas import tpu as pltpu
from jax.experimental.pallas import tpu_sc as plsc

V = 1000000
D = 64
B = 16384
NEG = 20
NP = NEG + 1

NC = 2
NS = 16
NW = NC * NS
B_PER_W = B // NW
BLK = 32
NBLK = B_PER_W // BLK
NG = BLK // 16

_CH = 8192
_NCH = (V + _CH - 1) // _CH
_VP = _NCH * _CH // 2
_HALF = _CH // 2
_SH_CH = _CH.bit_length() - 1
_SH_H = _HALF.bit_length() - 1


def _pad_body(x_ref, o_ref):
    proj = (
        lax.broadcasted_iota(jnp.int32, (D, D), 0)
        == lax.broadcasted_iota(jnp.int32, (D, D), 1)
    ).astype(jnp.float32)
    xt = lax.dot_general(
        x_ref[...], proj,
        dimension_numbers=(((0,), (0,)), ((), ())),
        preferred_element_type=jnp.float32,
    )
    o_ref[...] = jnp.concatenate([xt[: _CH // 2], xt[_CH // 2 :]], axis=1)


@jax.jit
def _tc_pad(tab_T):
    return pl.pallas_call(
        _pad_body,
        grid=(_NCH,),
        in_specs=[pl.BlockSpec((D, _CH), lambda i: (0, i))],
        out_specs=pl.BlockSpec((_CH // 2, 2 * D), lambda i: (i, 0)),
        out_shape=jax.ShapeDtypeStruct((_VP, 2 * D), jnp.float32),
    )(tab_T)


def _sc_body(cw_hbm, xw_hbm, nT_hbm, in_hbm, out_hbm,
             pos_hbm, negT_hbm,
             cidx, xidx, nidx, cpk, xpk, npk, crows, prows, scores, sem):
    wid = lax.axis_index("s") * NC + lax.axis_index("c")
    wbase = wid * B_PER_W
    lanes = lax.iota(jnp.int32, 16)

    pltpu.sync_copy(cw_hbm.at[pl.ds(wbase, B_PER_W)], cidx)
    pltpu.sync_copy(xw_hbm.at[pl.ds(wbase, B_PER_W)], xidx)
    for j in range(NEG):
        pltpu.sync_copy(nT_hbm.at[j, pl.ds(wbase, B_PER_W)], nidx.at[j])

    def pk_body(k, _):
        s = pl.ds(k * 16, 16)
        c, x = cidx[s], xidx[s]
        cpk[s] = ((c >> _SH_CH) << _SH_H) + (c & (_HALF - 1))
        xpk[s] = ((x >> _SH_CH) << _SH_H) + (x & (_HALF - 1))
        for j in range(NEG):
            nj = nidx[j, s]
            npk[j, s] = ((nj >> _SH_CH) << _SH_H) + (nj & (_HALF - 1))
        return _

    lax.fori_loop(0, B_PER_W // 16, pk_body, None)

    def blk_body(sb, _):
        off = sb * BLK
        descs = [
            pltpu.async_copy(in_hbm.at[cpk.at[pl.ds(off, BLK)]], crows, sem),
            pltpu.async_copy(out_hbm.at[xpk.at[pl.ds(off, BLK)]], prows.at[0], sem),
        ]
        for j in range(NEG):
            descs.append(
                pltpu.async_copy(
                    out_hbm.at[npk.at[j, pl.ds(off, BLK)]], prows.at[1 + j], sem
                )
            )
        for dsc in descs:
            dsc.wait()

        def group_body(g, _):
            rid = g * 16 + lanes
            s = pl.ds(off + g * 16, 16)
            cbase = ((cidx[s] >> _SH_H) & 1) * D
            pbase = [((xidx[s] >> _SH_H) & 1) * D] + [
                ((nidx[j, s] >> _SH_H) & 1) * D for j in range(NEG)
            ]

            def d_body(d, accs):
                cv = plsc.load_gather(crows, [rid, cbase + d])
                return tuple(
                    acc
                    + cv
                    * plsc.load_gather(
                        prows, [jnp.full((16,), t, jnp.int32), rid, pbase[t] + d]
                    )
                    for t, acc in enumerate(accs)
                )

            accs = lax.fori_loop(
                0, D, d_body, tuple(jnp.zeros((16,), jnp.float32) for _ in range(NP))
            )
            for t in range(NP):
                scores[t, pl.ds(g * 16, 16)] = accs[t]
            return _

        lax.fori_loop(0, NG, group_body, None)

        pltpu.sync_copy(scores.at[0], pos_hbm.at[pl.ds(wbase + off, BLK)])
        for j in range(NEG):
            pltpu.sync_copy(scores.at[1 + j], negT_hbm.at[j, pl.ds(wbase + off, BLK)])
        return _

    lax.fori_loop(0, NBLK, blk_body, None)


@jax.jit
def _sc_scores(center_words, context_words, neg_T, in_p, out_p):
    mesh = plsc.VectorSubcoreMesh(
        core_axis_name="c", subcore_axis_name="s", num_cores=NC, num_subcores=NS
    )
    f = pl.kernel(
        _sc_body,
        out_type=(
            jax.ShapeDtypeStruct((B,), jnp.float32),
            jax.ShapeDtypeStruct((NEG, B), jnp.float32),
        ),
        mesh=mesh,
        compiler_params=pltpu.CompilerParams(needs_layout_passes=False),
        scratch_types=[
            pltpu.VMEM((B_PER_W,), jnp.int32),
            pltpu.VMEM((B_PER_W,), jnp.int32),
            pltpu.VMEM((NEG, B_PER_W), jnp.int32),
            pltpu.VMEM((B_PER_W,), jnp.int32),
            pltpu.VMEM((B_PER_W,), jnp.int32),
            pltpu.VMEM((NEG, B_PER_W), jnp.int32),
            pltpu.VMEM((BLK, 2 * D), jnp.float32),
            pltpu.VMEM((NP, BLK, 2 * D), jnp.float32),
            pltpu.VMEM((NP, BLK), jnp.float32),
            pltpu.SemaphoreType.DMA,
        ],
    )
    return f(center_words, context_words, neg_T, in_p, out_p)


def _loss_body(pos_ref, neg_ref, out_ref):
    p = pos_ref[...]
    n = neg_ref[...]
    total = jnp.sum(jax.nn.log_sigmoid(p)) + jnp.sum(jax.nn.log_sigmoid(-n))
    out_ref[...] = jnp.reshape(-total / B, (1, 1))


@jax.jit
def _tc_loss(pos, neg):
    out = pl.pallas_call(
        _loss_body,
        out_shape=jax.ShapeDtypeStruct((1, 1), jnp.float32),
    )(pos.reshape(128, 128), neg.reshape(NEG * B // 128, 128))
    return out[0, 0]


def kernel(center_words, context_words, negative_words, in_emb, out_emb):
    in_p = _tc_pad(in_emb.T)
    out_p = _tc_pad(out_emb.T)
    neg_T = negative_words.T
    pos, negs = _sc_scores(center_words, context_words, neg_T, in_p, out_p)
    return _tc_loss(pos, negs)


# speedup vs baseline: 2.0376x; 1.0673x over previous
import jax
import jax.numpy as jnp
from jax import lax
from jax.experimental import pallas as pl
from jax.experimental.pallas import tpu as pltpu
from jax.experimental.pallas import tpu_sc as plsc

V = 1000000
D = 64
B = 16384
NEG = 20
NP = NEG + 1

NC = 2
NS = 16
NW = NC * NS
B_PER_W = B // NW
BLK = 32
NBLK = B_PER_W // BLK
NG = BLK // 16

_CH = 16384
_NCH = (V + _CH - 1) // _CH
_VP = _NCH * _CH // 2
_HALF = _CH // 2
_SH_CH = _CH.bit_length() - 1
_SH_H = _HALF.bit_length() - 1


def _pad_body(x_ref, o_ref):
    proj = (
        lax.broadcasted_iota(jnp.int32, (D, D), 0)
        == lax.broadcasted_iota(jnp.int32, (D, D), 1)
    ).astype(jnp.float32)
    xt = lax.dot_general(
        x_ref[...], proj,
        dimension_numbers=(((0,), (0,)), ((), ())),
        preferred_element_type=jnp.float32,
    )
    o_ref[...] = jnp.concatenate([xt[: _CH // 2], xt[_CH // 2 :]], axis=1)


@jax.jit
def _tc_pad(tab_T):
    return pl.pallas_call(
        _pad_body,
        grid=(_NCH,),
        in_specs=[pl.BlockSpec((D, _CH), lambda i: (0, i))],
        out_specs=pl.BlockSpec((_CH // 2, 2 * D), lambda i: (i, 0)),
        out_shape=jax.ShapeDtypeStruct((_VP, 2 * D), jnp.float32),
    )(tab_T)


def _sc_body(cw_hbm, xw_hbm, nT_hbm, in_hbm, out_hbm,
             pos_hbm, negT_hbm,
             cidx, xidx, nidx, cpk, xpk, npk, crows, prows, scores, sem):
    wid = lax.axis_index("s") * NC + lax.axis_index("c")
    wbase = wid * B_PER_W
    lanes = lax.iota(jnp.int32, 16)

    pltpu.sync_copy(cw_hbm.at[pl.ds(wbase, B_PER_W)], cidx)
    pltpu.sync_copy(xw_hbm.at[pl.ds(wbase, B_PER_W)], xidx)
    for j in range(NEG):
        pltpu.sync_copy(nT_hbm.at[j, pl.ds(wbase, B_PER_W)], nidx.at[j])

    def pk_body(k, _):
        s = pl.ds(k * 16, 16)
        c, x = cidx[s], xidx[s]
        cpk[s] = ((c >> _SH_CH) << _SH_H) + (c & (_HALF - 1))
        xpk[s] = ((x >> _SH_CH) << _SH_H) + (x & (_HALF - 1))
        for j in range(NEG):
            nj = nidx[j, s]
            npk[j, s] = ((nj >> _SH_CH) << _SH_H) + (nj & (_HALF - 1))
        return _

    lax.fori_loop(0, B_PER_W // 16, pk_body, None)

    def blk_body(sb, _):
        off = sb * BLK
        descs = [
            pltpu.async_copy(in_hbm.at[cpk.at[pl.ds(off, BLK)]], crows, sem),
            pltpu.async_copy(out_hbm.at[xpk.at[pl.ds(off, BLK)]], prows.at[0], sem),
        ]
        for j in range(NEG):
            descs.append(
                pltpu.async_copy(
                    out_hbm.at[npk.at[j, pl.ds(off, BLK)]], prows.at[1 + j], sem
                )
            )
        for dsc in descs:
            dsc.wait()

        def group_body(g, _):
            rid = g * 16 + lanes
            s = pl.ds(off + g * 16, 16)
            cbase = ((cidx[s] >> _SH_H) & 1) * D
            pbase = [((xidx[s] >> _SH_H) & 1) * D] + [
                ((nidx[j, s] >> _SH_H) & 1) * D for j in range(NEG)
            ]

            def d_body(d, accs):
                cv = plsc.load_gather(crows, [rid, cbase + d])
                return tuple(
                    acc
                    + cv
                    * plsc.load_gather(
                        prows, [jnp.full((16,), t, jnp.int32), rid, pbase[t] + d]
                    )
                    for t, acc in enumerate(accs)
                )

            accs = lax.fori_loop(
                0, D, d_body, tuple(jnp.zeros((16,), jnp.float32) for _ in range(NP))
            )
            for t in range(NP):
                scores[t, pl.ds(g * 16, 16)] = accs[t]
            return _

        lax.fori_loop(0, NG, group_body, None)

        pltpu.sync_copy(scores.at[0], pos_hbm.at[pl.ds(wbase + off, BLK)])
        for j in range(NEG):
            pltpu.sync_copy(scores.at[1 + j], negT_hbm.at[j, pl.ds(wbase + off, BLK)])
        return _

    lax.fori_loop(0, NBLK, blk_body, None)


@jax.jit
def _sc_scores(center_words, context_words, neg_T, in_p, out_p):
    mesh = plsc.VectorSubcoreMesh(
        core_axis_name="c", subcore_axis_name="s", num_cores=NC, num_subcores=NS
    )
    f = pl.kernel(
        _sc_body,
        out_type=(
            jax.ShapeDtypeStruct((B,), jnp.float32),
            jax.ShapeDtypeStruct((NEG, B), jnp.float32),
        ),
        mesh=mesh,
        compiler_params=pltpu.CompilerParams(needs_layout_passes=False),
        scratch_types=[
            pltpu.VMEM((B_PER_W,), jnp.int32),
            pltpu.VMEM((B_PER_W,), jnp.int32),
            pltpu.VMEM((NEG, B_PER_W), jnp.int32),
            pltpu.VMEM((B_PER_W,), jnp.int32),
            pltpu.VMEM((B_PER_W,), jnp.int32),
            pltpu.VMEM((NEG, B_PER_W), jnp.int32),
            pltpu.VMEM((BLK, 2 * D), jnp.float32),
            pltpu.VMEM((NP, BLK, 2 * D), jnp.float32),
            pltpu.VMEM((NP, BLK), jnp.float32),
            pltpu.SemaphoreType.DMA,
        ],
    )
    return f(center_words, context_words, neg_T, in_p, out_p)


def _loss_body(pos_ref, neg_ref, out_ref):
    p = pos_ref[...]
    n = neg_ref[...]
    total = jnp.sum(jax.nn.log_sigmoid(p)) + jnp.sum(jax.nn.log_sigmoid(-n))
    out_ref[...] = jnp.reshape(-total / B, (1, 1))


@jax.jit
def _tc_loss(pos, neg):
    out = pl.pallas_call(
        _loss_body,
        out_shape=jax.ShapeDtypeStruct((1, 1), jnp.float32),
    )(pos.reshape(128, 128), neg.reshape(NEG * B // 128, 128))
    return out[0, 0]


def kernel(center_words, context_words, negative_words, in_emb, out_emb):
    in_p = _tc_pad(in_emb.T)
    out_p = _tc_pad(out_emb.T)
    neg_T = negative_words.T
    pos, negs = _sc_scores(center_words, context_words, neg_T, in_p, out_p)
    return _tc_loss(pos, negs)


# speedup vs baseline: 2.0982x; 1.0297x over previous
import jax
import jax.numpy as jnp
from jax import lax
from jax.experimental import pallas as pl
from jax.experimental.pallas import tpu as pltpu
from jax.experimental.pallas import tpu_sc as plsc

V = 1000000
D = 64
B = 16384
NEG = 20
NP = NEG + 1

NC = 2
NS = 16
NW = NC * NS
B_PER_W = B // NW
BLK = 32
NBLK = B_PER_W // BLK
NG = BLK // 16

_CH = 32768
_NCH = (V + _CH - 1) // _CH
_VP = _NCH * _CH // 2
_HALF = _CH // 2
_SH_CH = _CH.bit_length() - 1
_SH_H = _HALF.bit_length() - 1


def _pad_body(x_ref, o_ref):
    proj = (
        lax.broadcasted_iota(jnp.int32, (D, D), 0)
        == lax.broadcasted_iota(jnp.int32, (D, D), 1)
    ).astype(jnp.float32)
    xt = lax.dot_general(
        x_ref[...], proj,
        dimension_numbers=(((0,), (0,)), ((), ())),
        preferred_element_type=jnp.float32,
    )
    o_ref[...] = jnp.concatenate([xt[: _CH // 2], xt[_CH // 2 :]], axis=1)


@jax.jit
def _tc_pad(tab_T):
    return pl.pallas_call(
        _pad_body,
        grid=(_NCH,),
        in_specs=[pl.BlockSpec((D, _CH), lambda i: (0, i))],
        out_specs=pl.BlockSpec((_CH // 2, 2 * D), lambda i: (i, 0)),
        out_shape=jax.ShapeDtypeStruct((_VP, 2 * D), jnp.float32),
    )(tab_T)


def _sc_body(cw_hbm, xw_hbm, nT_hbm, in_hbm, out_hbm,
             pos_hbm, negT_hbm,
             cidx, xidx, nidx, cpk, xpk, npk, crows, prows, scores, sem):
    wid = lax.axis_index("s") * NC + lax.axis_index("c")
    wbase = wid * B_PER_W
    lanes = lax.iota(jnp.int32, 16)

    pltpu.sync_copy(cw_hbm.at[pl.ds(wbase, B_PER_W)], cidx)
    pltpu.sync_copy(xw_hbm.at[pl.ds(wbase, B_PER_W)], xidx)
    for j in range(NEG):
        pltpu.sync_copy(nT_hbm.at[j, pl.ds(wbase, B_PER_W)], nidx.at[j])

    def pk_body(k, _):
        s = pl.ds(k * 16, 16)
        c, x = cidx[s], xidx[s]
        cpk[s] = ((c >> _SH_CH) << _SH_H) + (c & (_HALF - 1))
        xpk[s] = ((x >> _SH_CH) << _SH_H) + (x & (_HALF - 1))
        for j in range(NEG):
            nj = nidx[j, s]
            npk[j, s] = ((nj >> _SH_CH) << _SH_H) + (nj & (_HALF - 1))
        return _

    lax.fori_loop(0, B_PER_W // 16, pk_body, None)

    def blk_body(sb, _):
        off = sb * BLK
        descs = [
            pltpu.async_copy(in_hbm.at[cpk.at[pl.ds(off, BLK)]], crows, sem),
            pltpu.async_copy(out_hbm.at[xpk.at[pl.ds(off, BLK)]], prows.at[0], sem),
        ]
        for j in range(NEG):
            descs.append(
                pltpu.async_copy(
                    out_hbm.at[npk.at[j, pl.ds(off, BLK)]], prows.at[1 + j], sem
                )
            )
        for dsc in descs:
            dsc.wait()

        def group_body(g, _):
            rid = g * 16 + lanes
            s = pl.ds(off + g * 16, 16)
            cbase = ((cidx[s] >> _SH_H) & 1) * D
            pbase = [((xidx[s] >> _SH_H) & 1) * D] + [
                ((nidx[j, s] >> _SH_H) & 1) * D for j in range(NEG)
            ]

            def d_body(d, accs):
                cv = plsc.load_gather(crows, [rid, cbase + d])
                return tuple(
                    acc
                    + cv
                    * plsc.load_gather(
                        prows, [jnp.full((16,), t, jnp.int32), rid, pbase[t] + d]
                    )
                    for t, acc in enumerate(accs)
                )

            accs = lax.fori_loop(
                0, D, d_body, tuple(jnp.zeros((16,), jnp.float32) for _ in range(NP))
            )
            for t in range(NP):
                scores[t, pl.ds(g * 16, 16)] = accs[t]
            return _

        lax.fori_loop(0, NG, group_body, None)

        pltpu.sync_copy(scores.at[0], pos_hbm.at[pl.ds(wbase + off, BLK)])
        for j in range(NEG):
            pltpu.sync_copy(scores.at[1 + j], negT_hbm.at[j, pl.ds(wbase + off, BLK)])
        return _

    lax.fori_loop(0, NBLK, blk_body, None)


@jax.jit
def _sc_scores(center_words, context_words, neg_T, in_p, out_p):
    mesh = plsc.VectorSubcoreMesh(
        core_axis_name="c", subcore_axis_name="s", num_cores=NC, num_subcores=NS
    )
    f = pl.kernel(
        _sc_body,
        out_type=(
            jax.ShapeDtypeStruct((B,), jnp.float32),
            jax.ShapeDtypeStruct((NEG, B), jnp.float32),
        ),
        mesh=mesh,
        compiler_params=pltpu.CompilerParams(needs_layout_passes=False),
        scratch_types=[
            pltpu.VMEM((B_PER_W,), jnp.int32),
            pltpu.VMEM((B_PER_W,), jnp.int32),
            pltpu.VMEM((NEG, B_PER_W), jnp.int32),
            pltpu.VMEM((B_PER_W,), jnp.int32),
            pltpu.VMEM((B_PER_W,), jnp.int32),
            pltpu.VMEM((NEG, B_PER_W), jnp.int32),
            pltpu.VMEM((BLK, 2 * D), jnp.float32),
            pltpu.VMEM((NP, BLK, 2 * D), jnp.float32),
            pltpu.VMEM((NP, BLK), jnp.float32),
            pltpu.SemaphoreType.DMA,
        ],
    )
    return f(center_words, context_words, neg_T, in_p, out_p)


def _loss_body(pos_ref, neg_ref, out_ref):
    p = pos_ref[...]
    n = neg_ref[...]
    total = jnp.sum(jax.nn.log_sigmoid(p)) + jnp.sum(jax.nn.log_sigmoid(-n))
    out_ref[...] = jnp.reshape(-total / B, (1, 1))


@jax.jit
def _tc_loss(pos, neg):
    out = pl.pallas_call(
        _loss_body,
        out_shape=jax.ShapeDtypeStruct((1, 1), jnp.float32),
    )(pos.reshape(128, 128), neg.reshape(NEG * B // 128, 128))
    return out[0, 0]


def kernel(center_words, context_words, negative_words, in_emb, out_emb):
    in_p = _tc_pad(in_emb.T)
    out_p = _tc_pad(out_emb.T)
    neg_T = negative_words.T
    pos, negs = _sc_scores(center_words, context_words, neg_T, in_p, out_p)
    return _tc_loss(pos, negs)
